# baseline (speedup 1.0000x reference)
import functools

import jax
import jax.numpy as jnp
from jax import lax
from jax.experimental import pallas as pl
from jax.experimental.pallas import tpu as pltpu
from jax.experimental.pallas import tpu_sc as plsc

N_FLIGHT = 8000
N_AIRPORT = 2000
NV = 2000
NE = 160000
ROWS = 63
NROWS_OUT = 2016
NTILES = 32
CHUNK = 320
NBUF = 5
NCHUNK = NE // CHUNK
VPC = CHUNK // 16
VPR = NV // 16
BLK = 400


def _counts_body(src0, dst0, src1, dst1, c0_out, c1_out, acc,
                 sbufa, sbufb, sbufc, sbufd, sbufe,
                 dbufa, dbufb, dbufc, dbufd, dbufe,
                 ssem0, ssem1, ssem2, ssem3, ssem4,
                 dsem0, dsem1, dsem2, dsem3, dsem4, csem):
  wid = lax.axis_index("c") * 16 + lax.axis_index("s")
  lo = wid * ROWS
  sbufs = (sbufa, sbufb, sbufc, sbufd, sbufe)
  dbufs = (dbufa, dbufb, dbufc, dbufd, dbufe)
  ssems = (ssem0, ssem1, ssem2, ssem3, ssem4)
  dsems = (dsem0, dsem1, dsem2, dsem3, dsem4)

  def do_relation(src_hbm, dst_hbm, c_hbm):
    zero16 = jnp.zeros((16,), jnp.float32)

    @plsc.parallel_loop(0, ROWS * VPR, unroll=8)
    def _(i):
      acc[pl.ds(i * 16, 16)] = zero16

    def start(k, b):
      pltpu.async_copy(src_hbm.at[pl.ds(k * CHUNK, CHUNK)], sbufs[b],
                       ssems[b])
      pltpu.async_copy(dst_hbm.at[pl.ds(k * CHUNK, CHUNK)], dbufs[b],
                       dsems[b])

    def wait(b):
      pltpu.make_async_copy(src_hbm.at[pl.ds(0, CHUNK)], sbufs[b],
                            ssems[b]).wait()
      pltpu.make_async_copy(dst_hbm.at[pl.ds(0, CHUNK)], dbufs[b],
                            dsems[b]).wait()

    for b in range(NBUF):
      start(b, b)
    ones16 = jnp.full((16,), 1.0, jnp.float32)

    def chunk_group(p, carry):
      for b in range(NBUF):
        wait(b)

        @plsc.parallel_loop(0, VPC, unroll=5)
        def _(j):
          s = sbufs[b][pl.ds(j * 16, 16)]
          d = dbufs[b][pl.ds(j * 16, 16)]
          dr = d - lo
          m = jnp.logical_and(dr >= 0, dr < ROWS)
          flat = dr * NV + s
          plsc.addupdate_scatter(acc, [flat], ones16, mask=m)

        start(NBUF * p + b + NBUF, b)
      return carry

    lax.fori_loop(0, NCHUNK // NBUF, chunk_group, 0, unroll=False)
    for b in range(NBUF):
      wait(b)

    def norm_group(g, carry):
      @plsc.parallel_loop(9 * g, 9 * g + 9)
      def _(r):
        def sum_body(c, vs):
          base = r * NV + c * 80
          return tuple(
              vs[k] + acc[pl.ds(base + k * 16, 16)] for k in range(5))

        vs = lax.fori_loop(
            0, VPR // 5, sum_body,
            tuple(jnp.zeros((16,), jnp.float32) for _ in range(5)), unroll=2)
        v = (vs[0] + vs[1]) + (vs[2] + vs[3]) + vs[4]
        total = jnp.sum(v)
        t16 = jnp.full((16,), total, jnp.float32)
        scale = jnp.full((16,), 1.0, jnp.float32) / jnp.maximum(t16, 1.0)

        @plsc.parallel_loop(0, VPR, unroll=5)
        def _(c):
          sl = pl.ds(r * NV + c * 16, 16)
          acc[sl] = acc[sl] * scale

      pltpu.async_copy(acc.at[pl.ds(9 * g * NV, 9 * NV)],
                       c_hbm.at[pl.ds((lo + 9 * g) * NV, 9 * NV)], csem)
      return carry

    lax.fori_loop(0, ROWS // 9, norm_group, 0, unroll=False)
    for _ in range(ROWS // 9):
      pltpu.make_async_copy(acc.at[pl.ds(0, 9 * NV)],
                            c_hbm.at[pl.ds(0, 9 * NV)], csem).wait()

  do_relation(src0, dst0, c0_out)
  do_relation(src1, dst1, c1_out)


@functools.lru_cache(maxsize=1)
def _make_counts_kernel():
  mesh = plsc.VectorSubcoreMesh(core_axis_name="c", subcore_axis_name="s")
  return pl.kernel(
      _counts_body,
      out_type=(
          jax.ShapeDtypeStruct((NROWS_OUT * NV,), jnp.float32),
          jax.ShapeDtypeStruct((NROWS_OUT * NV,), jnp.float32),
      ),
      mesh=mesh,
      compiler_params=pltpu.CompilerParams(needs_layout_passes=False),
      scratch_types=(
          [pltpu.VMEM((ROWS * NV,), jnp.float32)]
          + [pltpu.VMEM((CHUNK,), jnp.int32)] * 10
          + [pltpu.SemaphoreType.DMA] * 11
      ),
  )


NB_ENC_F = N_FLIGHT // BLK
NB_ENC = 25
NB_L1_END = 50
NB1 = NV // BLK
GRID = 70


def _relu(x):
  return jnp.maximum(x, 0.0)


def _w(basis_ref, comp_ref, r):
  return (comp_ref[r, 0] * basis_ref[0:128, :]
          + comp_ref[r, 1] * basis_ref[128:256, :])


def _fused_body(xf_ref, wf_ref, bf_ref, xa_ref, wa_ref, ba_ref,
                root0_ref, bias0_ref, basis0_ref, comp0_ref,
                root1_ref, bias1_ref, basis1_ref, comp1_ref,
                c0_ref, c1_ref, wr_ref, br_ref, y_ref, x0s, x1s):
  s = pl.program_id(0)

  @pl.when(s < NB_ENC_F)
  def _():
    h = jnp.dot(xf_ref[...], wf_ref[...], preferred_element_type=jnp.float32)
    x0s[pl.ds(s * BLK, BLK), :] = _relu(h + bf_ref[...])

  @pl.when(jnp.logical_and(s >= NB_ENC_F, s < NB_ENC))
  def _():
    h = jnp.dot(xa_ref[...], wa_ref[...], preferred_element_type=jnp.float32)
    x0s[pl.ds(s * BLK, BLK), :] = _relu(h + ba_ref[...])

  i1 = s - NB_ENC

  @pl.when(jnp.logical_and(s >= NB_ENC, s < NB_ENC + NB1))
  def _():
    xblk = x0s[pl.ds(i1 * BLK, BLK), :]
    base = jnp.dot(xblk, root0_ref[...],
                   preferred_element_type=jnp.float32) + bias0_ref[...]
    agg = jnp.dot(c1_ref[...], x0s[pl.ds(N_FLIGHT, NV), :],
                  preferred_element_type=jnp.float32)
    msg = jnp.dot(agg, _w(basis0_ref, comp0_ref, 1),
                  preferred_element_type=jnp.float32)
    x1s[pl.ds(i1 * BLK, BLK), :] = _relu(base + msg)

  @pl.when(jnp.logical_and(s >= NB_ENC + NB1, s < NB_ENC + NB_ENC_F))
  def _():
    xblk = x0s[pl.ds(i1 * BLK, BLK), :]
    base = jnp.dot(xblk, root0_ref[...],
                   preferred_element_type=jnp.float32) + bias0_ref[...]
    x1s[pl.ds(i1 * BLK, BLK), :] = _relu(base)

  @pl.when(jnp.logical_and(s >= NB_ENC + NB_ENC_F, s < NB_L1_END))
  def _():
    xblk = x0s[pl.ds(i1 * BLK, BLK), :]
    base = jnp.dot(xblk, root0_ref[...],
                   preferred_element_type=jnp.float32) + bias0_ref[...]
    agg = jnp.dot(c0_ref[...], x0s[pl.ds(0, NV), :],
                  preferred_element_type=jnp.float32)
    msg = jnp.dot(agg, _w(basis0_ref, comp0_ref, 0),
                  preferred_element_type=jnp.float32)
    x1s[pl.ds(i1 * BLK, BLK), :] = _relu(base + msg)

  i2 = s - NB_L1_END

  @pl.when(jnp.logical_and(s >= NB_L1_END, s < NB_L1_END + NB1))
  def _():
    xblk = x1s[pl.ds(i2 * BLK, BLK), :]
    base = jnp.dot(xblk, root1_ref[...],
                   preferred_element_type=jnp.float32) + bias1_ref[...]
    agg = jnp.dot(c1_ref[...], x1s[pl.ds(N_FLIGHT, NV), :],
                  preferred_element_type=jnp.float32)
    msg = jnp.dot(agg, _w(basis1_ref, comp1_ref, 1),
                  preferred_element_type=jnp.float32)
    t = _relu(base + msg)
    y_ref[...] = jnp.dot(t, wr_ref[...],
                         preferred_element_type=jnp.float32) + br_ref[...]

  @pl.when(s >= NB_L1_END + NB1)
  def _():
    xblk = x1s[pl.ds(i2 * BLK, BLK), :]
    base = jnp.dot(xblk, root1_ref[...],
                   preferred_element_type=jnp.float32) + bias1_ref[...]
    t = _relu(base)
    y_ref[...] = jnp.dot(t, wr_ref[...],
                         preferred_element_type=jnp.float32) + br_ref[...]


def _fs(shape):
  return pl.BlockSpec(shape, lambda s: tuple(0 for _ in shape))


def _clip(v, lo, hi):
  return jnp.minimum(jnp.maximum(v, lo), hi)


def kernel(x_flight, x_airport, edge_index_f2a, edge_index_a2f,
           enc_flight_W, enc_flight_b, enc_airport_W, enc_airport_b,
           conv0_basis, conv0_comp, conv0_root, conv0_bias,
           conv1_basis, conv1_comp, conv1_root, conv1_bias,
           readout_W, readout_b):
  ei0 = edge_index_f2a.astype(jnp.int32)
  ei1 = edge_index_a2f.astype(jnp.int32)

  pad = NBUF * CHUNK
  c0_flat, c1_flat = _make_counts_kernel()(
      jnp.pad(ei0[0], (0, pad)), jnp.pad(ei0[1], (0, pad)),
      jnp.pad(ei1[0], (0, pad)), jnp.pad(ei1[1], (0, pad)))
  c0 = c0_flat.reshape(NROWS_OUT, NV)
  c1 = c1_flat.reshape(NROWS_OUT, NV)

  wrp = jnp.pad(readout_W, ((0, 0), (0, 127)))
  brp = jnp.pad(readout_b.reshape(1, 1), ((0, 0), (0, 127)))
  smem = pl.BlockSpec(memory_space=pltpu.SMEM)

  y = pl.pallas_call(
      _fused_body,
      grid=(GRID,),
      in_specs=[
          pl.BlockSpec((BLK, 128), lambda s: (_clip(s, 0, NB_ENC_F - 1), 0)),
          _fs((128, 128)), _fs((1, 128)),
          pl.BlockSpec((BLK, 128),
                       lambda s: (_clip(s - NB_ENC_F, 0, NB1 - 1), 0)),
          _fs((128, 128)), _fs((1, 128)),
          _fs((128, 128)), _fs((1, 128)), _fs((256, 128)), smem,
          _fs((128, 128)), _fs((1, 128)), _fs((256, 128)), smem,
          pl.BlockSpec((BLK, NV), lambda s: (_clip(s - 45, 0, NB1 - 1), 0)),
          pl.BlockSpec(
              (BLK, NV),
              lambda s: (jnp.where(s < NB_L1_END,
                                   _clip(s - NB_ENC, 0, NB1 - 1),
                                   _clip(s - NB_L1_END, 0, NB1 - 1)), 0)),
          _fs((128, 128)), _fs((1, 128)),
      ],
      out_specs=pl.BlockSpec((BLK, 128),
                             lambda s: (_clip(s - NB_L1_END, 0, 19), 0)),
      out_shape=jax.ShapeDtypeStruct((N_FLIGHT, 128), jnp.float32),
      scratch_shapes=[
          pltpu.VMEM((N_FLIGHT + NV, 128), jnp.float32),
          pltpu.VMEM((N_FLIGHT + NV, 128), jnp.float32),
      ],
  )(x_flight, enc_flight_W, enc_flight_b.reshape(1, 128),
    x_airport, enc_airport_W, enc_airport_b.reshape(1, 128),
    conv0_root, conv0_bias.reshape(1, 128), conv0_basis.reshape(256, 128),
    conv0_comp, conv1_root, conv1_bias.reshape(1, 128),
    conv1_basis.reshape(256, 128), conv1_comp, c0, c1, wrp, brp)

  return y[:, 0]


# speedup vs baseline: 1.0134x; 1.0134x over previous
import functools

import jax
import jax.numpy as jnp
from jax import lax
from jax.experimental import pallas as pl
from jax.experimental.pallas import tpu as pltpu
from jax.experimental.pallas import tpu_sc as plsc

N_FLIGHT = 8000
N_AIRPORT = 2000
NV = 2000
NE = 160000
ROWS = 63
NTILES = 32
CHUNK = 320
NBUF = 5
NCHUNK = NE // CHUNK
VPC = CHUNK // 16
VPR = NV // 16
BLK = 400


def _counts_body(src0, dst0, src1, dst1, c0_out, c1_out, acc,
                 sbufa, sbufb, sbufc, sbufd, sbufe,
                 dbufa, dbufb, dbufc, dbufd, dbufe,
                 ssem0, ssem1, ssem2, ssem3, ssem4,
                 dsem0, dsem1, dsem2, dsem3, dsem4):
  wid = lax.axis_index("c") * 16 + lax.axis_index("s")
  lo = wid * ROWS
  sbufs = (sbufa, sbufb, sbufc, sbufd, sbufe)
  dbufs = (dbufa, dbufb, dbufc, dbufd, dbufe)
  ssems = (ssem0, ssem1, ssem2, ssem3, ssem4)
  dsems = (dsem0, dsem1, dsem2, dsem3, dsem4)

  def do_relation(src_hbm, dst_hbm, c_hbm):
    zero16 = jnp.zeros((16,), jnp.float32)

    @plsc.parallel_loop(0, ROWS * VPR, unroll=8)
    def _(i):
      acc[pl.ds(i * 16, 16)] = zero16

    def start(k, b):
      pltpu.async_copy(src_hbm.at[pl.ds(k * CHUNK, CHUNK)], sbufs[b],
                       ssems[b])
      pltpu.async_copy(dst_hbm.at[pl.ds(k * CHUNK, CHUNK)], dbufs[b],
                       dsems[b])

    def wait(b):
      pltpu.make_async_copy(src_hbm.at[pl.ds(0, CHUNK)], sbufs[b],
                            ssems[b]).wait()
      pltpu.make_async_copy(dst_hbm.at[pl.ds(0, CHUNK)], dbufs[b],
                            dsems[b]).wait()

    for b in range(NBUF):
      start(b, b)
    ones16 = jnp.full((16,), 1.0, jnp.float32)

    def chunk_group(p, carry):
      for b in range(NBUF):
        wait(b)

        @plsc.parallel_loop(0, VPC, unroll=5)
        def _(j):
          s = sbufs[b][pl.ds(j * 16, 16)]
          d = dbufs[b][pl.ds(j * 16, 16)]
          dr = d - lo
          m = jnp.logical_and(dr >= 0, dr < ROWS)
          flat = dr * NV + s
          plsc.addupdate_scatter(acc, [flat], ones16, mask=m)

        @pl.when(NBUF * p + b + NBUF < NCHUNK)
        def _():
          start(NBUF * p + b + NBUF, b)
      return carry

    lax.fori_loop(0, NCHUNK // NBUF, chunk_group, 0, unroll=False)

    @plsc.parallel_loop(0, ROWS)
    def _(r):
      def sum_body(c, vs):
        base = r * NV + c * 80
        return tuple(
            vs[k] + acc[pl.ds(base + k * 16, 16)] for k in range(5))

      vs = lax.fori_loop(
          0, VPR // 5, sum_body,
          tuple(jnp.zeros((16,), jnp.float32) for _ in range(5)), unroll=2)
      v = (vs[0] + vs[1]) + (vs[2] + vs[3]) + vs[4]
      total = jnp.sum(v)
      t16 = jnp.full((16,), total, jnp.float32)
      scale = jnp.full((16,), 1.0, jnp.float32) / jnp.maximum(t16, 1.0)

      @plsc.parallel_loop(0, VPR, unroll=5)
      def _(c):
        sl = pl.ds(r * NV + c * 16, 16)
        acc[sl] = acc[sl] * scale

    tail = NV - ROWS * (NTILES - 1)

    @pl.when(wid < NTILES - 1)
    def _():
      pltpu.sync_copy(acc, c_hbm.at[pl.ds(lo * NV, ROWS * NV)])

    @pl.when(wid == NTILES - 1)
    def _():
      pltpu.sync_copy(acc.at[pl.ds(0, tail * NV)],
                      c_hbm.at[pl.ds(lo * NV, tail * NV)])

  do_relation(src0, dst0, c0_out)
  do_relation(src1, dst1, c1_out)


@functools.lru_cache(maxsize=1)
def _make_counts_kernel():
  mesh = plsc.VectorSubcoreMesh(core_axis_name="c", subcore_axis_name="s")
  return pl.kernel(
      _counts_body,
      out_type=(
          jax.ShapeDtypeStruct((NV * NV,), jnp.float32),
          jax.ShapeDtypeStruct((NV * NV,), jnp.float32),
      ),
      mesh=mesh,
      compiler_params=pltpu.CompilerParams(needs_layout_passes=False),
      scratch_types=(
          [pltpu.VMEM((ROWS * NV,), jnp.float32)]
          + [pltpu.VMEM((CHUNK,), jnp.int32)] * 10
          + [pltpu.SemaphoreType.DMA] * 10
      ),
  )


NB_ENC_F = N_FLIGHT // BLK
NB_ENC = 25
NB_L1_END = 50
NB1 = NV // BLK
GRID = 70


def _relu(x):
  return jnp.maximum(x, 0.0)


def _w(basis_ref, comp_ref, r):
  return (comp_ref[r, 0] * basis_ref[0:128, :]
          + comp_ref[r, 1] * basis_ref[128:256, :])


def _fused_body(xf_ref, wf_ref, bf_ref, xa_ref, wa_ref, ba_ref,
                root0_ref, bias0_ref, basis0_ref, comp0_ref,
                root1_ref, bias1_ref, basis1_ref, comp1_ref,
                c0_ref, c1_ref, wr_ref, br_ref, y_ref, x0s, x1s):
  s = pl.program_id(0)

  @pl.when(s < NB_ENC_F)
  def _():
    h = jnp.dot(xf_ref[...], wf_ref[...], preferred_element_type=jnp.float32)
    x0s[pl.ds(s * BLK, BLK), :] = _relu(h + bf_ref[...])

  @pl.when(jnp.logical_and(s >= NB_ENC_F, s < NB_ENC))
  def _():
    h = jnp.dot(xa_ref[...], wa_ref[...], preferred_element_type=jnp.float32)
    x0s[pl.ds(s * BLK, BLK), :] = _relu(h + ba_ref[...])

  i1 = s - NB_ENC

  @pl.when(jnp.logical_and(s >= NB_ENC, s < NB_ENC + NB1))
  def _():
    xblk = x0s[pl.ds(i1 * BLK, BLK), :]
    base = jnp.dot(xblk, root0_ref[...],
                   preferred_element_type=jnp.float32) + bias0_ref[...]
    agg = jnp.dot(c1_ref[...], x0s[pl.ds(N_FLIGHT, NV), :],
                  preferred_element_type=jnp.float32)
    msg = jnp.dot(agg, _w(basis0_ref, comp0_ref, 1),
                  preferred_element_type=jnp.float32)
    x1s[pl.ds(i1 * BLK, BLK), :] = _relu(base + msg)

  @pl.when(jnp.logical_and(s >= NB_ENC + NB1, s < NB_ENC + NB_ENC_F))
  def _():
    xblk = x0s[pl.ds(i1 * BLK, BLK), :]
    base = jnp.dot(xblk, root0_ref[...],
                   preferred_element_type=jnp.float32) + bias0_ref[...]
    x1s[pl.ds(i1 * BLK, BLK), :] = _relu(base)

  @pl.when(jnp.logical_and(s >= NB_ENC + NB_ENC_F, s < NB_L1_END))
  def _():
    xblk = x0s[pl.ds(i1 * BLK, BLK), :]
    base = jnp.dot(xblk, root0_ref[...],
                   preferred_element_type=jnp.float32) + bias0_ref[...]
    agg = jnp.dot(c0_ref[...], x0s[pl.ds(0, NV), :],
                  preferred_element_type=jnp.float32)
    msg = jnp.dot(agg, _w(basis0_ref, comp0_ref, 0),
                  preferred_element_type=jnp.float32)
    x1s[pl.ds(i1 * BLK, BLK), :] = _relu(base + msg)

  i2 = s - NB_L1_END

  @pl.when(jnp.logical_and(s >= NB_L1_END, s < NB_L1_END + NB1))
  def _():
    xblk = x1s[pl.ds(i2 * BLK, BLK), :]
    base = jnp.dot(xblk, root1_ref[...],
                   preferred_element_type=jnp.float32) + bias1_ref[...]
    agg = jnp.dot(c1_ref[...], x1s[pl.ds(N_FLIGHT, NV), :],
                  preferred_element_type=jnp.float32)
    msg = jnp.dot(agg, _w(basis1_ref, comp1_ref, 1),
                  preferred_element_type=jnp.float32)
    t = _relu(base + msg)
    y_ref[...] = jnp.dot(t, wr_ref[...],
                         preferred_element_type=jnp.float32) + br_ref[...]

  @pl.when(s >= NB_L1_END + NB1)
  def _():
    xblk = x1s[pl.ds(i2 * BLK, BLK), :]
    base = jnp.dot(xblk, root1_ref[...],
                   preferred_element_type=jnp.float32) + bias1_ref[...]
    t = _relu(base)
    y_ref[...] = jnp.dot(t, wr_ref[...],
                         preferred_element_type=jnp.float32) + br_ref[...]


def _fs(shape):
  return pl.BlockSpec(shape, lambda s: tuple(0 for _ in shape))


def _clip(v, lo, hi):
  return jnp.minimum(jnp.maximum(v, lo), hi)


def kernel(x_flight, x_airport, edge_index_f2a, edge_index_a2f,
           enc_flight_W, enc_flight_b, enc_airport_W, enc_airport_b,
           conv0_basis, conv0_comp, conv0_root, conv0_bias,
           conv1_basis, conv1_comp, conv1_root, conv1_bias,
           readout_W, readout_b):
  ei0 = edge_index_f2a.astype(jnp.int32)
  ei1 = edge_index_a2f.astype(jnp.int32)

  c0_flat, c1_flat = _make_counts_kernel()(ei0[0], ei0[1], ei1[0], ei1[1])
  c0 = c0_flat.reshape(NV, NV)
  c1 = c1_flat.reshape(NV, NV)

  wrp = jnp.pad(readout_W, ((0, 0), (0, 127)))
  brp = jnp.pad(readout_b.reshape(1, 1), ((0, 0), (0, 127)))
  smem = pl.BlockSpec(memory_space=pltpu.SMEM)

  y = pl.pallas_call(
      _fused_body,
      grid=(GRID,),
      in_specs=[
          pl.BlockSpec((BLK, 128), lambda s: (_clip(s, 0, NB_ENC_F - 1), 0)),
          _fs((128, 128)), _fs((1, 128)),
          pl.BlockSpec((BLK, 128),
                       lambda s: (_clip(s - NB_ENC_F, 0, NB1 - 1), 0)),
          _fs((128, 128)), _fs((1, 128)),
          _fs((128, 128)), _fs((1, 128)), _fs((256, 128)), smem,
          _fs((128, 128)), _fs((1, 128)), _fs((256, 128)), smem,
          pl.BlockSpec((BLK, NV), lambda s: (_clip(s - 45, 0, NB1 - 1), 0)),
          pl.BlockSpec(
              (BLK, NV),
              lambda s: (jnp.where(s < NB_L1_END,
                                   _clip(s - NB_ENC, 0, NB1 - 1),
                                   _clip(s - NB_L1_END, 0, NB1 - 1)), 0)),
          _fs((128, 128)), _fs((1, 128)),
      ],
      out_specs=pl.BlockSpec((BLK, 128),
                             lambda s: (_clip(s - NB_L1_END, 0, 19), 0)),
      out_shape=jax.ShapeDtypeStruct((N_FLIGHT, 128), jnp.float32),
      scratch_shapes=[
          pltpu.VMEM((N_FLIGHT + NV, 128), jnp.float32),
          pltpu.VMEM((N_FLIGHT + NV, 128), jnp.float32),
      ],
  )(x_flight, enc_flight_W, enc_flight_b.reshape(1, 128),
    x_airport, enc_airport_W, enc_airport_b.reshape(1, 128),
    conv0_root, conv0_bias.reshape(1, 128), conv0_basis.reshape(256, 128),
    conv0_comp, conv1_root, conv1_bias.reshape(1, 128),
    conv1_basis.reshape(256, 128), conv1_comp, c0, c1, wrp, brp)

  return y[:, 0]
